# baseline (speedup 1.0000x reference)
import functools
import jax
import jax.numpy as jnp
from jax import lax
from jax.experimental import pallas as pl
from jax.experimental.pallas import tpu as pltpu
from jax.experimental.pallas import tpu_sc as plsc

L = 16

_GATHER_DN = lax.GatherDimensionNumbers(
    offset_dims=(), collapsed_slice_dims=(0,), start_index_map=(0,))


def _splat(v, j):
    idx = jnp.full((L, 1), j, dtype=jnp.int32)
    return lax.gather(v, idx, _GATHER_DN, (1,),
                      mode=lax.GatherScatterMode.PROMISE_IN_BOUNDS)


def _rsqrt16(x):
    i = lax.bitcast_convert_type(x, jnp.int32)
    i = jnp.full((L,), 0x5F3759DF, dtype=jnp.int32) - lax.shift_right_logical(
        i, jnp.full((L,), 1, dtype=jnp.int32))
    y = lax.bitcast_convert_type(i, jnp.float32)
    half = jnp.full((L,), 0.5, dtype=jnp.float32)
    three_half = jnp.full((L,), 1.5, dtype=jnp.float32)
    hx = half * x
    for _ in range(2):
        y = y * (three_half - hx * y * y)
    return y


def _make_kernel(B, S, H, V, eps):
    assert H == 4 * L
    info = plsc.get_sparse_core_info()
    nw = info.num_cores * info.num_subcores
    bw = B // nw
    assert B == nw * 128 and bw == 128
    assert S % 2 == 0
    ntiles = B // 128
    hg_n = H // 8

    mesh = plsc.VectorSubcoreMesh(core_axis_name="c", subcore_axis_name="s")

    @functools.partial(
        pl.kernel,
        out_type=jax.ShapeDtypeStruct((B * S * H,), jnp.float32),
        mesh=mesh,
        scratch_types=[
            pltpu.VMEM((bw, S), jnp.int32),
            pltpu.VMEM((bw, S), jnp.int32),
            pltpu.VMEM((2, bw), jnp.int32),
            pltpu.VMEM((2, bw, H), jnp.float32),
            pltpu.VMEM((2, 8 * 1024), jnp.float32),
            pltpu.VMEM((S * H,), jnp.float32),
            pltpu.VMEM((2 * H,), jnp.float32),
            pltpu.VMEM((H,), jnp.float32),
            pltpu.VMEM((H,), jnp.float32),
            pltpu.VMEM((L * L,), jnp.float32),
            pltpu.VMEM((L * L,), jnp.float32),
            pltpu.SemaphoreType.DMA,
            pltpu.SemaphoreType.DMA,
            pltpu.SemaphoreType.DMA,
            pltpu.SemaphoreType.DMA,
        ],
        compiler_params=pltpu.CompilerParams(
            use_tc_tiling_on_sc=False, needs_layout_passes=False),
    )
    def k(ids_hbm, tt_hbm, tok_table_hbm, pos_hbm, typ_hbm, g_hbm, b_hbm,
          out_hbm, ids_all, tt_all, idx_s, tok_s, stage, pos_v, typ_v,
          g_v, b_v, ssum_v, sq_v, sem_gat0, sem_gat1, sem_out0, sem_out1):
        sem_gat = (sem_gat0, sem_gat1)
        sem_out = (sem_out0, sem_out1)
        wid = lax.axis_index("s") * info.num_cores + lax.axis_index("c")
        b0 = wid * bw

        pltpu.sync_copy(ids_hbm.at[pl.ds(b0, bw)], ids_all)
        pltpu.sync_copy(tt_hbm.at[pl.ds(b0, bw)], tt_all)
        pltpu.sync_copy(pos_hbm, pos_v)
        pltpu.sync_copy(typ_hbm, typ_v)
        pltpu.sync_copy(g_hbm, g_v)
        pltpu.sync_copy(b_hbm, b_v)

        t0 = [typ_v[pl.ds(g * L, L)] for g in range(4)]
        td = [typ_v[pl.ds(H + g * L, L)] - t0[g] for g in range(4)]
        gam = [g_v[pl.ds(g * L, L)] for g in range(4)]
        bet = [b_v[pl.ds(g * L, L)] for g in range(4)]

        def fold(s, carry):
            for g in range(4):
                sl = pl.ds(s * H + g * L, L)
                pos_v[sl] = pos_v[sl] + t0[g]
            return carry

        lax.fori_loop(0, S, fold, 0, unroll=4)

        inv_h = jnp.float32(1.0 / H)
        iota16 = lax.iota(jnp.int32, L)
        iota256 = iota16 * L
        pat = (lax.shift_right_logical(iota16, jnp.full((L,), 3, jnp.int32))
               * 1024) + (
            (iota16 & jnp.full((L,), 7, jnp.int32)) * 128)

        def compact_ids(sv, b):
            svec = lax.broadcast(sv, (L,))
            for k2 in range(bw // L):
                rows = iota16 + k2 * L
                vals = plsc.load_gather(ids_all, [rows, svec])
                idx_s[b, pl.ds(k2 * L, L)] = vals

        def gat_copy(b):
            return pltpu.make_async_copy(
                tok_table_hbm.at[idx_s.at[b]], tok_s.at[b], sem_gat[b])

        def out_copies(sv, b):
            res = []
            for hg in range(hg_n):
                base = ((sv * hg_n + hg) * ntiles + wid) * 1024
                res.append(pltpu.make_async_copy(
                    stage.at[b, pl.ds(hg * 1024, 1024)],
                    out_hbm.at[pl.ds(base, 1024)], sem_out[b]))
            return res

        def compute(sv, b):
            tokb = tok_s.at[b]
            stageb = stage.at[b]
            svec = lax.broadcast(sv, (L,))
            pos_row = [pos_v[pl.ds(sv * H + g * L, L)] for g in range(4)]
            def per_chunk(c8, carry2):
                tbase = c8 * L
                tt16 = plsc.load_gather(
                    tt_all, [iota16 + tbase, svec]).astype(jnp.float32)
                for j in range(L):
                    tl = tbase + j
                    ttf = _splat(tt16, j)
                    e = []
                    for g in range(4):
                        e.append(tokb[tl, pl.ds(g * L, L)] + pos_row[g]
                                 + ttf * td[g])
                    sum4 = (e[0] + e[1]) + (e[2] + e[3])
                    q4 = (e[0] * e[0] + e[1] * e[1]) + (
                        e[2] * e[2] + e[3] * e[3])
                    ssum_v[pl.ds(j * L, L)] = sum4
                    sq_v[pl.ds(j * L, L)] = q4
                    idxb = pat + tl
                    for g in range(4):
                        plsc.store_scatter(stageb, [idxb + g * 2048], e[g])
                acc_s = plsc.load_gather(ssum_v, [iota256])
                acc_q = plsc.load_gather(sq_v, [iota256])
                for l in range(1, L):
                    idx = iota256 + l
                    acc_s = acc_s + plsc.load_gather(ssum_v, [idx])
                    acc_q = acc_q + plsc.load_gather(sq_v, [idx])
                mean = acc_s * inv_h
                var = acc_q * inv_h - mean * mean
                rstd = _rsqrt16(var + eps)
                for j in range(L):
                    tl = tbase + j
                    m_s = _splat(mean, j)
                    r_s = _splat(rstd, j)
                    idxb = pat + tl
                    for g in range(4):
                        idx = idxb + g * 2048
                        ev = plsc.load_gather(stageb, [idx])
                        plsc.store_scatter(
                            stageb, [idx],
                            (ev - m_s) * (r_s * gam[g]) + bet[g])
                return carry2

            lax.fori_loop(0, bw // L, per_chunk, 0)

        compact_ids(0, 0)
        gat_copy(0).start()
        compact_ids(1, 1)
        gat_copy(1).start()

        def outer(i, carry):
            s2 = i * 2
            for b in (0, 1):
                sv = s2 + b
                gat_copy(b).wait()

                @pl.when(sv >= 2)
                def _():
                    for c in out_copies(sv - 2, b):
                        c.wait()

                compute(sv, b)

                @pl.when(sv + 2 < S)
                def _():
                    compact_ids(sv + 2, b)
                    gat_copy(b).start()

                for c in out_copies(sv, b):
                    c.start()
            return carry

        lax.fori_loop(0, S // 2, outer, 0)
        for c in out_copies(S - 2, 0):
            c.wait()
        for c in out_copies(S - 1, 1):
            c.wait()

    return k


def kernel(input_ids, token_type_ids, token_table, pos_table, type_table,
           gamma, beta):
    B, S = input_ids.shape
    V, H = token_table.shape
    eps = jnp.float32(1e-5)
    k = _make_kernel(B, S, H, V, eps)
    out_flat = k(
        input_ids,
        token_type_ids,
        token_table,
        pos_table[:S].reshape(-1),
        type_table.reshape(-1),
        gamma,
        beta,
    )
    out5 = out_flat.reshape(S, H // 8, B // 128, 8, 128)
    return out5.transpose(2, 4, 0, 1, 3).reshape(B, S, H)


# speedup vs baseline: 1.9194x; 1.9194x over previous
import functools
import jax
import jax.numpy as jnp
from jax import lax
from jax.experimental import pallas as pl
from jax.experimental.pallas import tpu as pltpu
from jax.experimental.pallas import tpu_sc as plsc

L = 16

_GATHER_DN = lax.GatherDimensionNumbers(
    offset_dims=(), collapsed_slice_dims=(0,), start_index_map=(0,))


def _splat(v, j):
    idx = jnp.full((L, 1), j, dtype=jnp.int32)
    return lax.gather(v, idx, _GATHER_DN, (1,),
                      mode=lax.GatherScatterMode.PROMISE_IN_BOUNDS)


def _rsqrt16(x):
    i = lax.bitcast_convert_type(x, jnp.int32)
    i = jnp.full((L,), 0x5F3759DF, dtype=jnp.int32) - lax.shift_right_logical(
        i, jnp.full((L,), 1, dtype=jnp.int32))
    y = lax.bitcast_convert_type(i, jnp.float32)
    half = jnp.full((L,), 0.5, dtype=jnp.float32)
    three_half = jnp.full((L,), 1.5, dtype=jnp.float32)
    hx = half * x
    for _ in range(2):
        y = y * (three_half - hx * y * y)
    return y


def _make_kernel(B, S, H, V, eps):
    assert H == 4 * L
    info = plsc.get_sparse_core_info()
    nw = info.num_cores * info.num_subcores
    bw = B // nw
    assert B == nw * 128 and bw == 128
    assert S % 2 == 0
    ntiles = B // 128
    hg_n = H // 8

    mesh = plsc.VectorSubcoreMesh(core_axis_name="c", subcore_axis_name="s")

    @functools.partial(
        pl.kernel,
        out_type=jax.ShapeDtypeStruct((B * S * H,), jnp.float32),
        mesh=mesh,
        scratch_types=[
            pltpu.VMEM((bw, S), jnp.int32),
            pltpu.VMEM((bw, S), jnp.int32),
            pltpu.VMEM((2, bw), jnp.int32),
            pltpu.VMEM((2, bw, H), jnp.float32),
            pltpu.VMEM((2, 8 * 1024), jnp.float32),
            pltpu.VMEM((S * H,), jnp.float32),
            pltpu.VMEM((2 * H,), jnp.float32),
            pltpu.VMEM((H,), jnp.float32),
            pltpu.VMEM((H,), jnp.float32),
            pltpu.VMEM((L * L,), jnp.float32),
            pltpu.VMEM((L * L,), jnp.float32),
            pltpu.SemaphoreType.DMA,
            pltpu.SemaphoreType.DMA,
            pltpu.SemaphoreType.DMA,
            pltpu.SemaphoreType.DMA,
        ],
        compiler_params=pltpu.CompilerParams(
            use_tc_tiling_on_sc=False, needs_layout_passes=False),
    )
    def k(ids_hbm, tt_hbm, tok_table_hbm, pos_hbm, typ_hbm, g_hbm, b_hbm,
          out_hbm, ids_all, tt_all, idx_s, tok_s, stage, pos_v, typ_v,
          g_v, b_v, ssum_v, sq_v, sem_gat0, sem_gat1, sem_out0, sem_out1):
        sem_gat = (sem_gat0, sem_gat1)
        sem_out = (sem_out0, sem_out1)
        wid = lax.axis_index("s") * info.num_cores + lax.axis_index("c")
        b0 = wid * bw

        pltpu.sync_copy(ids_hbm.at[pl.ds(b0, bw)], ids_all)
        pltpu.sync_copy(tt_hbm.at[pl.ds(b0, bw)], tt_all)
        pltpu.sync_copy(pos_hbm, pos_v)
        pltpu.sync_copy(typ_hbm, typ_v)
        pltpu.sync_copy(g_hbm, g_v)
        pltpu.sync_copy(b_hbm, b_v)

        t0 = [typ_v[pl.ds(g * L, L)] for g in range(4)]
        td = [typ_v[pl.ds(H + g * L, L)] - t0[g] for g in range(4)]
        gam = [g_v[pl.ds(g * L, L)] for g in range(4)]
        bet = [b_v[pl.ds(g * L, L)] for g in range(4)]

        def fold(s, carry):
            for g in range(4):
                sl = pl.ds(s * H + g * L, L)
                pos_v[sl] = pos_v[sl] + t0[g]
            return carry

        lax.fori_loop(0, S, fold, 0, unroll=4)

        inv_h = jnp.float32(1.0 / H)
        iota16 = lax.iota(jnp.int32, L)
        iota256 = iota16 * L
        pat = (lax.shift_right_logical(iota16, jnp.full((L,), 3, jnp.int32))
               * 1024) + (
            (iota16 & jnp.full((L,), 7, jnp.int32)) * 128)

        def compact_ids(sv, b):
            svec = lax.broadcast(sv, (L,))
            for k2 in range(bw // L):
                rows = iota16 + k2 * L
                vals = plsc.load_gather(ids_all, [rows, svec])
                idx_s[b, pl.ds(k2 * L, L)] = vals

        def gat_copy(b):
            return pltpu.make_async_copy(
                tok_table_hbm.at[idx_s.at[b]], tok_s.at[b], sem_gat[b])

        def out_copies(sv, b):
            res = []
            for hg in range(hg_n):
                base = ((sv * hg_n + hg) * ntiles + wid) * 1024
                res.append(pltpu.make_async_copy(
                    stage.at[b, pl.ds(hg * 1024, 1024)],
                    out_hbm.at[pl.ds(base, 1024)], sem_out[b]))
            return res

        def compute(sv, b):
            tokb = tok_s.at[b]
            stageb = stage.at[b]
            svec = lax.broadcast(sv, (L,))
            pos_row = [pos_v[pl.ds(sv * H + g * L, L)] for g in range(4)]
            def per_chunk(c8, carry2):
                tbase = c8 * L
                tt16 = plsc.load_gather(
                    tt_all, [iota16 + tbase, svec]).astype(jnp.float32)
                def emb(j):
                    tl = tbase + j
                    ttf = _splat(tt16, j)
                    return [tokb[tl, pl.ds(g * L, L)] + pos_row[g]
                            + ttf * td[g] for g in range(4)]

                for j in range(L):
                    e = emb(j)
                    sum4 = (e[0] + e[1]) + (e[2] + e[3])
                    q4 = (e[0] * e[0] + e[1] * e[1]) + (
                        e[2] * e[2] + e[3] * e[3])
                    ssum_v[pl.ds(j * L, L)] = sum4
                    sq_v[pl.ds(j * L, L)] = q4
                cs = [plsc.load_gather(ssum_v, [iota256 + l]) for l in range(L)]
                cq = [plsc.load_gather(sq_v, [iota256 + l]) for l in range(L)]
                while len(cs) > 1:
                    cs = [cs[i] + cs[i + 1] for i in range(0, len(cs), 2)]
                    cq = [cq[i] + cq[i + 1] for i in range(0, len(cq), 2)]
                mean = cs[0] * inv_h
                var = cq[0] * inv_h - mean * mean
                rstd = _rsqrt16(var + eps)
                for j in range(L):
                    tl = tbase + j
                    e = emb(j)
                    m_s = _splat(mean, j)
                    rg = _splat(rstd, j)
                    idxb = pat + tl
                    for g in range(4):
                        plsc.store_scatter(
                            stageb, [idxb + g * 2048],
                            (e[g] - m_s) * (rg * gam[g]) + bet[g])
                return carry2

            lax.fori_loop(0, bw // L, per_chunk, 0)

        compact_ids(0, 0)
        gat_copy(0).start()
        compact_ids(1, 1)
        gat_copy(1).start()

        def outer(i, carry):
            s2 = i * 2
            for b in (0, 1):
                sv = s2 + b
                gat_copy(b).wait()

                @pl.when(sv >= 2)
                def _():
                    for c in out_copies(sv - 2, b):
                        c.wait()

                compute(sv, b)

                @pl.when(sv + 2 < S)
                def _():
                    compact_ids(sv + 2, b)
                    gat_copy(b).start()

                for c in out_copies(sv, b):
                    c.start()
            return carry

        lax.fori_loop(0, S // 2, outer, 0)
        for c in out_copies(S - 2, 0):
            c.wait()
        for c in out_copies(S - 1, 1):
            c.wait()

    return k


def kernel(input_ids, token_type_ids, token_table, pos_table, type_table,
           gamma, beta):
    B, S = input_ids.shape
    V, H = token_table.shape
    eps = jnp.float32(1e-5)
    k = _make_kernel(B, S, H, V, eps)
    out_flat = k(
        input_ids,
        token_type_ids,
        token_table,
        pos_table[:S].reshape(-1),
        type_table.reshape(-1),
        gamma,
        beta,
    )
    out5 = out_flat.reshape(S, H // 8, B // 128, 8, 128)
    return out5.transpose(2, 4, 0, 1, 3).reshape(B, S, H)


# speedup vs baseline: 2.5772x; 1.3427x over previous
import functools
import jax
import jax.numpy as jnp
from jax import lax
from jax.experimental import pallas as pl
from jax.experimental.pallas import tpu as pltpu
from jax.experimental.pallas import tpu_sc as plsc

L = 16

_GATHER_DN = lax.GatherDimensionNumbers(
    offset_dims=(), collapsed_slice_dims=(0,), start_index_map=(0,))


def _splat(v, j):
    idx = jnp.full((L, 1), j, dtype=jnp.int32)
    return lax.gather(v, idx, _GATHER_DN, (1,),
                      mode=lax.GatherScatterMode.PROMISE_IN_BOUNDS)


def _rsqrt16(x):
    i = lax.bitcast_convert_type(x, jnp.int32)
    i = jnp.full((L,), 0x5F3759DF, dtype=jnp.int32) - lax.shift_right_logical(
        i, jnp.full((L,), 1, dtype=jnp.int32))
    y = lax.bitcast_convert_type(i, jnp.float32)
    half = jnp.full((L,), 0.5, dtype=jnp.float32)
    three_half = jnp.full((L,), 1.5, dtype=jnp.float32)
    hx = half * x
    for _ in range(2):
        y = y * (three_half - hx * y * y)
    return y


def _make_kernel(B, S, H, V, eps):
    assert H == 4 * L
    info = plsc.get_sparse_core_info()
    nw = info.num_cores * info.num_subcores
    bw = B // nw
    assert B == nw * 128 and bw == 128
    assert S % 2 == 0
    ntiles = B // 128
    hg_n = H // 8

    mesh = plsc.VectorSubcoreMesh(core_axis_name="c", subcore_axis_name="s")

    @functools.partial(
        pl.kernel,
        out_type=jax.ShapeDtypeStruct((B * S * H,), jnp.float32),
        mesh=mesh,
        scratch_types=[
            pltpu.VMEM((bw, S), jnp.int32),
            pltpu.VMEM((bw, S), jnp.int32),
            pltpu.VMEM((2, bw), jnp.int32),
            pltpu.VMEM((2, bw, H), jnp.float32),
            pltpu.VMEM((2, 8 * 1024), jnp.float32),
            pltpu.VMEM((S * H,), jnp.float32),
            pltpu.VMEM((2 * H,), jnp.float32),
            pltpu.VMEM((H,), jnp.float32),
            pltpu.VMEM((H,), jnp.float32),
            pltpu.VMEM((L * 17,), jnp.float32),
            pltpu.VMEM((L * 17,), jnp.float32),
            pltpu.VMEM((L * 65,), jnp.float32),
            pltpu.SemaphoreType.DMA,
            pltpu.SemaphoreType.DMA,
            pltpu.SemaphoreType.DMA,
            pltpu.SemaphoreType.DMA,
        ],
        compiler_params=pltpu.CompilerParams(
            use_tc_tiling_on_sc=False, needs_layout_passes=False),
    )
    def k(ids_hbm, tt_hbm, tok_table_hbm, pos_hbm, typ_hbm, g_hbm, b_hbm,
          out_hbm, ids_all, tt_all, idx_s, tok_s, stage, pos_v, typ_v,
          g_v, b_v, ssum_v, sq_v, tr_v,
          sem_gat0, sem_gat1, sem_out0, sem_out1):
        sem_gat = (sem_gat0, sem_gat1)
        sem_out = (sem_out0, sem_out1)
        wid = lax.axis_index("s") * info.num_cores + lax.axis_index("c")
        b0 = wid * bw

        pltpu.sync_copy(ids_hbm.at[pl.ds(b0, bw)], ids_all)
        pltpu.sync_copy(tt_hbm.at[pl.ds(b0, bw)], tt_all)
        pltpu.sync_copy(pos_hbm, pos_v)
        pltpu.sync_copy(typ_hbm, typ_v)
        pltpu.sync_copy(g_hbm, g_v)
        pltpu.sync_copy(b_hbm, b_v)

        t0 = [typ_v[pl.ds(g * L, L)] for g in range(4)]
        td = [typ_v[pl.ds(H + g * L, L)] - t0[g] for g in range(4)]
        gam = [g_v[pl.ds(g * L, L)] for g in range(4)]
        bet = [b_v[pl.ds(g * L, L)] for g in range(4)]

        def fold(s, carry):
            for g in range(4):
                sl = pl.ds(s * H + g * L, L)
                pos_v[sl] = pos_v[sl] + t0[g]
            return carry

        lax.fori_loop(0, S, fold, 0, unroll=4)

        inv_h = jnp.float32(1.0 / H)
        iota16 = lax.iota(jnp.int32, L)
        iota17 = iota16 * 17
        iota65 = iota16 * 65

        def compact_ids(sv, b):
            svec = lax.broadcast(sv, (L,))
            for k2 in range(bw // L):
                rows = iota16 + k2 * L
                vals = plsc.load_gather(ids_all, [rows, svec])
                idx_s[b, pl.ds(k2 * L, L)] = vals

        def gat_copy(b):
            return pltpu.make_async_copy(
                tok_table_hbm.at[idx_s.at[b]], tok_s.at[b], sem_gat[b])

        def out_copies(sv, b):
            res = []
            for hg in range(hg_n):
                base = ((sv * hg_n + hg) * ntiles + wid) * 1024
                res.append(pltpu.make_async_copy(
                    stage.at[b, pl.ds(hg * 1024, 1024)],
                    out_hbm.at[pl.ds(base, 1024)], sem_out[b]))
            return res

        def compute(sv, b):
            tokb = tok_s.at[b]
            stageb = stage.at[b]
            svec = lax.broadcast(sv, (L,))
            pos_row = [pos_v[pl.ds(sv * H + g * L, L)] for g in range(4)]
            def per_chunk(c8, carry2):
                tbase = c8 * L
                tt16 = plsc.load_gather(
                    tt_all, [iota16 + tbase, svec]).astype(jnp.float32)
                def emb(j):
                    tl = tbase + j
                    ttf = _splat(tt16, j)
                    return [tokb[tl, pl.ds(g * L, L)] + pos_row[g]
                            + ttf * td[g] for g in range(4)]

                for j in range(L):
                    e = emb(j)
                    sum4 = (e[0] + e[1]) + (e[2] + e[3])
                    q4 = (e[0] * e[0] + e[1] * e[1]) + (
                        e[2] * e[2] + e[3] * e[3])
                    ssum_v[pl.ds(j * 17, L)] = sum4
                    sq_v[pl.ds(j * 17, L)] = q4
                cs = [plsc.load_gather(ssum_v, [iota17 + l]) for l in range(L)]
                cq = [plsc.load_gather(sq_v, [iota17 + l]) for l in range(L)]
                while len(cs) > 1:
                    cs = [cs[i] + cs[i + 1] for i in range(0, len(cs), 2)]
                    cq = [cq[i] + cq[i + 1] for i in range(0, len(cq), 2)]
                mean = cs[0] * inv_h
                var = cq[0] * inv_h - mean * mean
                rstd = _rsqrt16(var + eps)
                for j in range(L):
                    e = emb(j)
                    m_s = _splat(mean, j)
                    rg = _splat(rstd, j)
                    for g in range(4):
                        tr_v[pl.ds(j * 65 + g * L, L)] = (
                            (e[g] - m_s) * (rg * gam[g]) + bet[g])
                for h in range(H):
                    col = plsc.load_gather(tr_v, [iota65 + h])
                    stageb[pl.ds((h // 8) * 1024 + (h % 8) * 128 + tbase,
                                 L)] = col
                return carry2

            lax.fori_loop(0, bw // L, per_chunk, 0)

        compact_ids(0, 0)
        gat_copy(0).start()
        compact_ids(1, 1)
        gat_copy(1).start()

        def outer(i, carry):
            s2 = i * 2
            for b in (0, 1):
                sv = s2 + b
                gat_copy(b).wait()

                @pl.when(sv >= 2)
                def _():
                    for c in out_copies(sv - 2, b):
                        c.wait()

                compute(sv, b)

                @pl.when(sv + 2 < S)
                def _():
                    compact_ids(sv + 2, b)
                    gat_copy(b).start()

                for c in out_copies(sv, b):
                    c.start()
            return carry

        lax.fori_loop(0, S // 2, outer, 0)
        for c in out_copies(S - 2, 0):
            c.wait()
        for c in out_copies(S - 1, 1):
            c.wait()

    return k


def kernel(input_ids, token_type_ids, token_table, pos_table, type_table,
           gamma, beta):
    B, S = input_ids.shape
    V, H = token_table.shape
    eps = jnp.float32(1e-5)
    k = _make_kernel(B, S, H, V, eps)
    out_flat = k(
        input_ids,
        token_type_ids,
        token_table,
        pos_table[:S].reshape(-1),
        type_table.reshape(-1),
        gamma,
        beta,
    )
    out5 = out_flat.reshape(S, H // 8, B // 128, 8, 128)
    return out5.transpose(2, 4, 0, 1, 3).reshape(B, S, H)


# speedup vs baseline: 3.0118x; 1.1686x over previous
import functools
import jax
import jax.numpy as jnp
from jax import lax
from jax.experimental import pallas as pl
from jax.experimental.pallas import tpu as pltpu
from jax.experimental.pallas import tpu_sc as plsc

L = 16

_GATHER_DN = lax.GatherDimensionNumbers(
    offset_dims=(), collapsed_slice_dims=(0,), start_index_map=(0,))


def _splat(v, j):
    idx = jnp.full((L, 1), j, dtype=jnp.int32)
    return lax.gather(v, idx, _GATHER_DN, (1,),
                      mode=lax.GatherScatterMode.PROMISE_IN_BOUNDS)


def _rsqrt16(x):
    i = lax.bitcast_convert_type(x, jnp.int32)
    i = jnp.full((L,), 0x5F3759DF, dtype=jnp.int32) - lax.shift_right_logical(
        i, jnp.full((L,), 1, dtype=jnp.int32))
    y = lax.bitcast_convert_type(i, jnp.float32)
    half = jnp.full((L,), 0.5, dtype=jnp.float32)
    three_half = jnp.full((L,), 1.5, dtype=jnp.float32)
    hx = half * x
    for _ in range(2):
        y = y * (three_half - hx * y * y)
    return y


def _make_kernel(B, S, H, V, eps):
    assert H == 4 * L
    info = plsc.get_sparse_core_info()
    nw = info.num_cores * info.num_subcores
    bw = B // nw
    assert B == nw * 128 and bw == 128
    assert S % 2 == 0
    ntiles = B // 128
    hg_n = H // 8

    mesh = plsc.VectorSubcoreMesh(core_axis_name="c", subcore_axis_name="s")

    @functools.partial(
        pl.kernel,
        out_type=jax.ShapeDtypeStruct((B * S * H,), jnp.float32),
        mesh=mesh,
        scratch_types=[
            pltpu.VMEM((bw, S), jnp.int32),
            pltpu.VMEM((bw, S), jnp.int32),
            pltpu.VMEM((2, bw), jnp.int32),
            pltpu.VMEM((2, bw, H), jnp.float32),
            pltpu.VMEM((2, 8 * 1024), jnp.float32),
            pltpu.VMEM((S * H,), jnp.float32),
            pltpu.VMEM((2 * H,), jnp.float32),
            pltpu.VMEM((H,), jnp.float32),
            pltpu.VMEM((H,), jnp.float32),
            pltpu.VMEM((L * 17,), jnp.float32),
            pltpu.VMEM((L * 17,), jnp.float32),
            pltpu.VMEM((L * 65,), jnp.float32),
            pltpu.SemaphoreType.DMA,
            pltpu.SemaphoreType.DMA,
            pltpu.SemaphoreType.DMA,
            pltpu.SemaphoreType.DMA,
        ],
        compiler_params=pltpu.CompilerParams(
            use_tc_tiling_on_sc=False, needs_layout_passes=False),
    )
    def k(ids_hbm, tt_hbm, tok_table_hbm, pos_hbm, typ_hbm, g_hbm, b_hbm,
          out_hbm, ids_all, tt_all, idx_s, tok_s, stage, pos_v, typ_v,
          g_v, b_v, ssum_v, sq_v, tr_v,
          sem_gat0, sem_gat1, sem_out0, sem_out1):
        sem_gat = (sem_gat0, sem_gat1)
        sem_out = (sem_out0, sem_out1)
        wid = lax.axis_index("s") * info.num_cores + lax.axis_index("c")
        b0 = wid * bw

        pltpu.sync_copy(ids_hbm.at[pl.ds(b0, bw)], ids_all)
        pltpu.sync_copy(tt_hbm.at[pl.ds(b0, bw)], tt_all)
        pltpu.sync_copy(pos_hbm, pos_v)
        pltpu.sync_copy(typ_hbm, typ_v)
        pltpu.sync_copy(g_hbm, g_v)
        pltpu.sync_copy(b_hbm, b_v)

        t0 = [typ_v[pl.ds(g * L, L)] for g in range(4)]
        td = [typ_v[pl.ds(H + g * L, L)] - t0[g] for g in range(4)]
        gam = [g_v[pl.ds(g * L, L)] for g in range(4)]
        bet = [b_v[pl.ds(g * L, L)] for g in range(4)]

        def fold(s, carry):
            for g in range(4):
                sl = pl.ds(s * H + g * L, L)
                pos_v[sl] = pos_v[sl] + t0[g]
            return carry

        lax.fori_loop(0, S, fold, 0, unroll=4)

        inv_h = jnp.float32(1.0 / H)
        iota16 = lax.iota(jnp.int32, L)
        iota17 = iota16 * 17
        iota65 = iota16 * 65

        def compact_ids(sv, b):
            svec = lax.broadcast(sv, (L,))
            for k2 in range(bw // L):
                rows = iota16 + k2 * L
                vals = plsc.load_gather(ids_all, [rows, svec])
                idx_s[b, pl.ds(k2 * L, L)] = vals

        def gat_copy(b):
            return pltpu.make_async_copy(
                tok_table_hbm.at[idx_s.at[b]], tok_s.at[b], sem_gat[b])

        def out_copies(sv, b):
            res = []
            for hg in range(hg_n):
                base = ((sv * hg_n + hg) * ntiles + wid) * 1024
                res.append(pltpu.make_async_copy(
                    stage.at[b, pl.ds(hg * 1024, 1024)],
                    out_hbm.at[pl.ds(base, 1024)], sem_out[b]))
            return res

        def compute(sv, b):
            tokb = tok_s.at[b]
            stageb = stage.at[b]
            svec = lax.broadcast(sv, (L,))
            pos_row = [pos_v[pl.ds(sv * H + g * L, L)] for g in range(4)]
            def per_chunk(c8, carry2):
                tbase = c8 * L
                tt16 = plsc.load_gather(
                    tt_all, [iota16 + tbase, svec]).astype(jnp.float32)
                def emb(j):
                    tl = tbase + j
                    ttf = _splat(tt16, j)
                    return [tokb[tl, pl.ds(g * L, L)] + pos_row[g]
                            + ttf * td[g] for g in range(4)]

                for j in range(L):
                    e = emb(j)
                    sum4 = (e[0] + e[1]) + (e[2] + e[3])
                    q4 = (e[0] * e[0] + e[1] * e[1]) + (
                        e[2] * e[2] + e[3] * e[3])
                    ssum_v[pl.ds(j * 17, L)] = sum4
                    sq_v[pl.ds(j * 17, L)] = q4
                    for g in range(4):
                        tr_v[pl.ds(j * 65 + g * L, L)] = e[g]
                cs = [plsc.load_gather(ssum_v, [iota17 + l]) for l in range(L)]
                cq = [plsc.load_gather(sq_v, [iota17 + l]) for l in range(L)]
                while len(cs) > 1:
                    cs = [cs[i] + cs[i + 1] for i in range(0, len(cs), 2)]
                    cq = [cq[i] + cq[i + 1] for i in range(0, len(cq), 2)]
                mean = cs[0] * inv_h
                var = cq[0] * inv_h - mean * mean
                rstd = _rsqrt16(var + eps)
                for j in range(L):
                    m_s = _splat(mean, j)
                    rg = _splat(rstd, j)
                    for g in range(4):
                        sl = pl.ds(j * 65 + g * L, L)
                        tr_v[sl] = (tr_v[sl] - m_s) * (rg * gam[g]) + bet[g]
                for h in range(H):
                    col = plsc.load_gather(tr_v, [iota65 + h])
                    stageb[pl.ds((h // 8) * 1024 + (h % 8) * 128 + tbase,
                                 L)] = col
                return carry2

            lax.fori_loop(0, bw // L, per_chunk, 0)

        compact_ids(0, 0)
        gat_copy(0).start()
        compact_ids(1, 1)
        gat_copy(1).start()

        def outer(i, carry):
            s2 = i * 2
            for b in (0, 1):
                sv = s2 + b
                gat_copy(b).wait()

                @pl.when(sv >= 2)
                def _():
                    for c in out_copies(sv - 2, b):
                        c.wait()

                compute(sv, b)

                @pl.when(sv + 2 < S)
                def _():
                    compact_ids(sv + 2, b)
                    gat_copy(b).start()

                for c in out_copies(sv, b):
                    c.start()
            return carry

        lax.fori_loop(0, S // 2, outer, 0)
        for c in out_copies(S - 2, 0):
            c.wait()
        for c in out_copies(S - 1, 1):
            c.wait()

    return k


def kernel(input_ids, token_type_ids, token_table, pos_table, type_table,
           gamma, beta):
    B, S = input_ids.shape
    V, H = token_table.shape
    eps = jnp.float32(1e-5)
    k = _make_kernel(B, S, H, V, eps)
    out_flat = k(
        input_ids,
        token_type_ids,
        token_table,
        pos_table[:S].reshape(-1),
        type_table.reshape(-1),
        gamma,
        beta,
    )
    out5 = out_flat.reshape(S, H // 8, B // 128, 8, 128)
    return out5.transpose(2, 4, 0, 1, 3).reshape(B, S, H)
